# Initial kernel scaffold; baseline (speedup 1.0000x reference)
import jax
import jax.numpy as jnp
from jax.experimental import pallas as pl


def kernel(type_indices, type_embedding_weight):
    raise NotImplementedError("write your pallas kernel here")


# speedup vs baseline: 9.1989x; 9.1989x over previous
import functools

import jax
import jax.numpy as jnp
from jax import lax
from jax.experimental import pallas as pl
from jax.experimental.pallas import tpu as pltpu
from jax.experimental.pallas import tpu_sc as plsc

_V = 1000
_D = 16
_B = 3_200_000

_info = plsc.get_sparse_core_info()
_NC = _info.num_cores
_NS = _info.num_subcores
_NW = _NC * _NS
_BPW = _B // _NW
_C = 2000
_NCHUNK = _BPW // _C

_mesh = plsc.VectorSubcoreMesh(core_axis_name="c", subcore_axis_name="s")


@functools.partial(
    pl.kernel,
    mesh=_mesh,
    out_type=jax.ShapeDtypeStruct((_B, _D), jnp.float32),
    scratch_types=[
        pltpu.VMEM((_C,), jnp.int32),
        pltpu.VMEM((_C,), jnp.int32),
        pltpu.VMEM((_C, _D), jnp.float32),
        pltpu.VMEM((_C, _D), jnp.float32),
        pltpu.VMEM_SHARED((_V, _D), jnp.float32),
        pltpu.SemaphoreType.DMA,
        pltpu.SemaphoreType.DMA,
        pltpu.SemaphoreType.DMA,
        pltpu.SemaphoreType.DMA,
        pltpu.SemaphoreType.DMA,
    ],
    compiler_params=pltpu.CompilerParams(use_tc_tiling_on_sc=False),
)
def _lookup(idx_hbm, table_hbm, out_hbm, idx_a, idx_b, rows_a, rows_b,
            table_sh, si0, si1, sg, so0, so1):
    idx = (idx_a, idx_b)
    rows = (rows_a, rows_b)
    si = (si0, si1)
    so = (so0, so1)
    sid = lax.axis_index("s")
    wid = sid * _NC + lax.axis_index("c")
    base = wid * _BPW

    @pl.when(sid == 0)
    def _():
        pltpu.sync_copy(table_hbm, table_sh)

    plsc.subcore_barrier()

    def istart(g, b):
        pltpu.async_copy(idx_hbm.at[pl.ds(base + g * _C, _C)], idx[b], si[b])

    def iwait(g, b):
        pltpu.make_async_copy(
            idx_hbm.at[pl.ds(base + g * _C, _C)], idx[b], si[b]
        ).wait()

    def ostart(g, b):
        pltpu.async_copy(rows[b], out_hbm.at[pl.ds(base + g * _C, _C)], so[b])

    def owait(g, b):
        pltpu.make_async_copy(
            rows[b], out_hbm.at[pl.ds(base + g * _C, _C)], so[b]
        ).wait()

    istart(0, 0)
    istart(1, 1)

    def pair(gp, carry):
        g0 = gp * 2

        @pl.when(gp > 0)
        def _():
            owait(g0 - 2, 0)
        iwait(g0, 0)
        pltpu.async_copy(table_sh.at[idx[0]], rows[0], sg).wait()
        ostart(g0, 0)

        @pl.when(g0 + 2 < _NCHUNK)
        def _():
            istart(g0 + 2, 0)

        @pl.when(gp > 0)
        def _():
            owait(g0 - 1, 1)
        iwait(g0 + 1, 1)
        pltpu.async_copy(table_sh.at[idx[1]], rows[1], sg).wait()
        ostart(g0 + 1, 1)

        @pl.when(g0 + 3 < _NCHUNK)
        def _():
            istart(g0 + 3, 1)
        return carry

    lax.fori_loop(0, _NCHUNK // 2, pair, 0)
    owait(_NCHUNK - 2, 0)
    owait(_NCHUNK - 1, 1)


def kernel(type_indices, type_embedding_weight):
    return _lookup(type_indices, type_embedding_weight)
